# baseline (speedup 1.0000x reference)
import numpy as np
import jax
import jax.numpy as jnp
from jax import lax
from jax.experimental import pallas as pl
from jax.experimental.pallas import tpu as pltpu

B, K, V = 64, 10, 100000
NROWS = B * K

_K0 = np.uint32(0)
_K1 = np.uint32(42)
_KS = (_K0, _K1, np.uint32(_K0 ^ _K1 ^ np.uint32(0x1BD11BDA)))
_ROT = (13, 15, 26, 6, 17, 29, 16, 24, 13, 15, 26, 6, 17, 29, 16, 24, 13, 15, 26, 6)
_TINY = np.float32(np.finfo(np.float32).tiny)
_NEG_INF = np.float32(-np.inf)

ROWS_PER_BLOCK = 8
GRID_R = NROWS // ROWS_PER_BLOCK
CHUNK = 512
UNROLL = 2
SB = 10240
NSB = -(-V // SB)
NCHUNK = SB // CHUNK


def _i32(x):
    return np.int32(np.uint32(x))


def _sample_kernel(w_ref, out_ref, vmax_ref, vtag_ref):
    pid = pl.program_id(0)
    sb = pl.program_id(1)
    shape = (ROWS_PER_BLOCK, CHUNK)
    row = lax.broadcasted_iota(jnp.int32, shape, 0) + pid * ROWS_PER_BLOCK
    lane = lax.broadcasted_iota(jnp.int32, shape, 1)
    tagb = row * V + lane + (sb * SB + _i32(_KS[1]))
    lanelim = V - sb * SB

    def rotl(x, r):
        return lax.shift_left(x, _i32(r)) | lax.shift_right_logical(x, _i32(32 - r))

    def chunk_score(c):
        w = w_ref[:, pl.ds(pl.multiple_of(c * CHUNK, 128), CHUNK)]
        tag = tagb + c * CHUNK
        x1 = tag
        x0 = x1
        x1 = rotl(x1, _ROT[0])
        x1 = x0 ^ x1
        for j in range(1, 4):
            x0 = x0 + x1
            x1 = rotl(x1, _ROT[j])
            x1 = x0 ^ x1
        x0 = x0 + _i32(_KS[1])
        x1 = x1 + _i32(np.uint32(_KS[2]) + np.uint32(1))
        for g in range(1, 5):
            for j in range(4):
                x0 = x0 + x1
                x1 = rotl(x1, _ROT[g * 4 + j])
                x1 = x0 ^ x1
            x0 = x0 + _i32(_KS[(g + 1) % 3])
            x1 = x1 + _i32(np.uint32(_KS[(g + 2) % 3]) + np.uint32(g + 1))
        bits = x0 ^ x1

        fb = lax.shift_right_logical(bits, _i32(9)) | _i32(0x3F800000)
        u = lax.bitcast_convert_type(fb, jnp.float32) - np.float32(1.0)
        u = jnp.maximum(u, _TINY)
        g = -jnp.log(-jnp.log(u))
        s = jnp.log(w + np.float32(1e-20)) + g
        s = jnp.where(lane + c * CHUNK < lanelim, s, _NEG_INF)
        return s, tag

    def update(carry, s, tag):
        vmax, vtag = carry
        upd = s > vmax
        vmax = jnp.maximum(vmax, s)
        vtag = jnp.where(upd, tag, vtag)
        return vmax, vtag

    def body(i, carry):
        for k in range(UNROLL):
            carry = update(carry, *chunk_score(i * UNROLL + k))
        return carry

    @pl.when(sb == 0)
    def _():
        vmax_ref[...] = jnp.full(shape, _NEG_INF, jnp.float32)
        vtag_ref[...] = jnp.zeros(shape, jnp.int32)

    carry = (vmax_ref[...], vtag_ref[...])
    carry = lax.fori_loop(0, NCHUNK // UNROLL, body, carry)
    vmax, vtag = carry
    vmax_ref[...] = vmax
    vtag_ref[...] = vtag

    @pl.when(sb == NSB - 1)
    def _():
        m = jnp.max(vmax, axis=1, keepdims=True)
        sel = jnp.where(vmax == m, vtag, np.int32(np.iinfo(np.int32).max))
        best_tag = jnp.min(sel, axis=1, keepdims=True)
        rowv = (lax.broadcasted_iota(jnp.int32, (ROWS_PER_BLOCK, 1), 0)
                + pid * ROWS_PER_BLOCK) * V + _i32(_KS[1])
        out_ref[0, 0, :] = (best_tag - rowv)[:, 0]


@jax.jit
def kernel(batch_k_head_softmax):
    w = batch_k_head_softmax.reshape(NROWS, V)
    out = pl.pallas_call(
        _sample_kernel,
        grid=(GRID_R, NSB),
        in_specs=[pl.BlockSpec((ROWS_PER_BLOCK, SB), lambda i, j: (i, j))],
        out_specs=pl.BlockSpec((1, 1, ROWS_PER_BLOCK), lambda i, j: (i, 0, 0)),
        out_shape=jax.ShapeDtypeStruct((GRID_R, 1, ROWS_PER_BLOCK), jnp.int32),
        scratch_shapes=[
            pltpu.VMEM((ROWS_PER_BLOCK, CHUNK), jnp.float32),
            pltpu.VMEM((ROWS_PER_BLOCK, CHUNK), jnp.int32),
        ],
        compiler_params=pltpu.CompilerParams(
            dimension_semantics=("parallel", "arbitrary"),
        ),
    )(w)
    return out.reshape(B, K)
